# Initial kernel scaffold; baseline (speedup 1.0000x reference)
import jax
import jax.numpy as jnp
from jax import lax
from jax.experimental import pallas as pl
from jax.experimental.pallas import tpu as pltpu
from jax.experimental.pallas import tpu_sc as plsc

N = 10000
E = 320000
D = 128

NC = 2
NS = 16
LANES = 16

C = 80
EPT = E // (NC * NS)
NCHUNK = EPT // C
RPS = N // NS
ZR = 125


def _sc_agg_body(h_hbm, src_hbm, dst_hbm, lap_hbm, out_hbm,
                 acc, src_v, dst_v, lap_v, rows_v, zero_v, sem):
    c = lax.axis_index("c")
    s = lax.axis_index("s")
    tile = c * NS + s

    def zrow(i, _):
        for k in range(D // LANES):
            zero_v[i, pl.ds(k * LANES, LANES)] = jnp.zeros((LANES,), jnp.float32)
        return 0
    lax.fori_loop(0, ZR, zrow, 0)
    for k in range(RPS // ZR):
        pltpu.sync_copy(zero_v, acc.at[pl.ds(s * RPS + k * ZR, ZR)])
    plsc.subcore_barrier()

    r0 = tile * NCHUNK
    pltpu.sync_copy(src_hbm.at[pl.ds(r0, NCHUNK)], src_v)
    pltpu.sync_copy(dst_hbm.at[pl.ds(r0, NCHUNK)], dst_v)
    pltpu.sync_copy(lap_hbm.at[pl.ds(r0, NCHUNK)], lap_v)

    def chunk(i, _):
        pltpu.async_copy(h_hbm.at[src_v.at[i]], rows_v, sem).wait()

        def group(g, _):
            off = pl.multiple_of(g * LANES, LANES)
            lvec = lap_v[i, pl.ds(off, LANES)]
            for j in range(LANES):
                lj = jnp.take(lvec, jnp.full((LANES,), j, jnp.int32),
                              mode="promise_in_bounds")
                e = off + j
                for k in range(D // LANES):
                    rows_v[e, pl.ds(k * LANES, LANES)] = (
                        rows_v[e, pl.ds(k * LANES, LANES)] * lj)
            return 0
        lax.fori_loop(0, C // LANES, group, 0)

        pltpu.sync_copy(rows_v, acc.at[dst_v.at[i]], add=True)
        return 0
    lax.fori_loop(0, NCHUNK, chunk, 0)

    plsc.subcore_barrier()
    for k in range(RPS // ZR):
        row = s * RPS + k * ZR
        pltpu.sync_copy(acc.at[pl.ds(row, ZR)], out_hbm.at[c, pl.ds(row, ZR)])


_sc_aggregate = pl.kernel(
    _sc_agg_body,
    out_type=jax.ShapeDtypeStruct((NC, N, D), jnp.float32),
    mesh=plsc.VectorSubcoreMesh(core_axis_name="c", subcore_axis_name="s"),
    scratch_types=[
        pltpu.VMEM_SHARED((N, D), jnp.float32),
        pltpu.VMEM((NCHUNK, C), jnp.int32),
        pltpu.VMEM((NCHUNK, C), jnp.int32),
        pltpu.VMEM((NCHUNK, C), jnp.float32),
        pltpu.VMEM((C, D), jnp.float32),
        pltpu.VMEM((ZR, D), jnp.float32),
        pltpu.SemaphoreType.DMA,
    ],
)


def _tc_update_body(h_ref, parts_ref, dinv_ref, w_ref, b_ref, out_ref):
    agg = parts_ref[0] + parts_ref[1]
    hh = h_ref[...] + dinv_ref[...] * agg
    y = jnp.dot(hh, w_ref[...], preferred_element_type=jnp.float32) + b_ref[...]
    out_ref[...] = jnp.maximum(y, 0.0)


def _tc_update(h, parts, d_inv2, w, b2):
    blk = 400
    grid = (N // blk,)
    return pl.pallas_call(
        _tc_update_body,
        grid=grid,
        in_specs=[
            pl.BlockSpec((blk, D), lambda i: (i, 0)),
            pl.BlockSpec((NC, blk, D), lambda i: (0, i, 0)),
            pl.BlockSpec((blk, 1), lambda i: (i, 0)),
            pl.BlockSpec((D, D), lambda i: (0, 0)),
            pl.BlockSpec((1, D), lambda i: (0, 0)),
        ],
        out_specs=pl.BlockSpec((blk, D), lambda i: (i, 0)),
        out_shape=jax.ShapeDtypeStruct((N, D), jnp.float32),
    )(h, parts, d_inv2, w, b2)


@jax.jit
def kernel(x, edge_index, lap, d_inv, W0, b0, W2, b2):
    src2 = edge_index[0].reshape(E // C, C)
    dst2 = edge_index[1].reshape(E // C, C)
    lap2 = lap.reshape(E // C, C)
    d_inv2 = d_inv[:, None]
    b0_2 = b0[None, :]
    b2_2 = b2[None, :]

    h = x
    for w, b in ((W0, b0_2), (W0, b0_2), (W0, b0_2), (W2, b2_2)):
        parts = _sc_aggregate(h, src2, dst2, lap2)
        h = _tc_update(h, parts, d_inv2, w, b)
    return h


# speedup vs baseline: 6.1290x; 6.1290x over previous
import jax
import jax.numpy as jnp
from jax import lax
from jax.experimental import pallas as pl
from jax.experimental.pallas import tpu as pltpu
from jax.experimental.pallas import tpu_sc as plsc

N = 10000
E = 320000
D = 128

NC = 2
NS = 16
LANES = 16

C = 80
EPT = E // (NC * NS)
NCHUNK = EPT // C
RPS = 624


def _lane_broadcast(vec, j):
    idx = jnp.full((LANES, 1), j, jnp.int32)
    dnums = lax.GatherDimensionNumbers(
        offset_dims=(), collapsed_slice_dims=(0,), start_index_map=(0,))
    return lax.gather(vec, idx, dnums, (1,),
                      mode=lax.GatherScatterMode.PROMISE_IN_BOUNDS)


def _sc_agg_body(h_hbm, src_hbm, dst_hbm, lap_hbm, out_hbm,
                 acc, src_v, dst_v, lap_v, rows_v, sem):
    c = lax.axis_index("c")
    s = lax.axis_index("s")
    tile = c * NS + s

    def zrow(i, _):
        for k in range(D // LANES):
            rows_v[i, pl.ds(k * LANES, LANES)] = jnp.zeros((LANES,), jnp.float32)
        return 0
    lax.fori_loop(0, C, zrow, 0)
    for k in range(RPS // C):
        pltpu.sync_copy(rows_v, acc.at[pl.ds(s * RPS + k * C, C)])
    pltpu.sync_copy(rows_v.at[pl.ds(0, 64)],
                    acc.at[pl.ds(s * RPS + (RPS // C) * C, 64)])

    @pl.when(s == NS - 1)
    def _():
        pltpu.sync_copy(rows_v.at[pl.ds(0, 16)], acc.at[pl.ds(N - 16, 16)])
    plsc.subcore_barrier()

    pltpu.sync_copy(src_hbm.at[pl.ds(tile * EPT, EPT)], src_v)
    pltpu.sync_copy(dst_hbm.at[tile], dst_v)
    pltpu.sync_copy(lap_hbm.at[pl.ds(tile * EPT, EPT)], lap_v)

    def chunk(i, _):
        pltpu.async_copy(h_hbm.at[src_v.at[pl.ds(i * C, C)]], rows_v, sem).wait()

        def group(g, _):
            off = pl.multiple_of(g * LANES, LANES)
            lvec = lap_v[pl.ds(pl.multiple_of(i * C + off, LANES), LANES)]
            for j in range(LANES):
                lj = _lane_broadcast(lvec, j)
                e = off + j
                for k in range(D // LANES):
                    rows_v[e, pl.ds(k * LANES, LANES)] = (
                        rows_v[e, pl.ds(k * LANES, LANES)] * lj)
            return 0
        lax.fori_loop(0, C // LANES, group, 0)

        pltpu.sync_copy(rows_v, acc.at[dst_v.at[i]], add=True)
        return 0
    lax.fori_loop(0, NCHUNK, chunk, 0)

    plsc.subcore_barrier()
    for k in range(2):
        row = s * RPS + k * 312
        pltpu.sync_copy(acc.at[pl.ds(row, 312)], out_hbm.at[c, pl.ds(row, 312)])

    @pl.when(s == NS - 1)
    def _():
        pltpu.sync_copy(acc.at[pl.ds(N - 16, 16)],
                        out_hbm.at[c, pl.ds(N - 16, 16)])


_sc_aggregate = pl.kernel(
    _sc_agg_body,
    out_type=jax.ShapeDtypeStruct((NC, N, D), jnp.float32),
    mesh=plsc.VectorSubcoreMesh(core_axis_name="c", subcore_axis_name="s"),
    scratch_types=[
        pltpu.VMEM_SHARED((N, D), jnp.float32),
        pltpu.VMEM((EPT,), jnp.int32),
        pltpu.VMEM((NCHUNK, C), jnp.int32),
        pltpu.VMEM((EPT,), jnp.float32),
        pltpu.VMEM((C, D), jnp.float32),
        pltpu.SemaphoreType.DMA,
    ],
)


def _tc_update_body(h_ref, parts_ref, dinv_ref, w_ref, b_ref, out_ref):
    agg = parts_ref[0] + parts_ref[1]
    hh = h_ref[...] + dinv_ref[...] * agg
    y = jnp.dot(hh, w_ref[...], preferred_element_type=jnp.float32) + b_ref[...]
    out_ref[...] = jnp.maximum(y, 0.0)


def _tc_update(h, parts, d_inv2, w, b2):
    blk = 400
    grid = (N // blk,)
    return pl.pallas_call(
        _tc_update_body,
        grid=grid,
        in_specs=[
            pl.BlockSpec((blk, D), lambda i: (i, 0)),
            pl.BlockSpec((NC, blk, D), lambda i: (0, i, 0)),
            pl.BlockSpec((blk, 1), lambda i: (i, 0)),
            pl.BlockSpec((D, D), lambda i: (0, 0)),
            pl.BlockSpec((1, D), lambda i: (0, 0)),
        ],
        out_specs=pl.BlockSpec((blk, D), lambda i: (i, 0)),
        out_shape=jax.ShapeDtypeStruct((N, D), jnp.float32),
    )(h, parts, d_inv2, w, b2)


@jax.jit
def kernel(x, edge_index, lap, d_inv, W0, b0, W2, b2):
    nt = NC * NS
    src2 = edge_index[0]
    dst2 = edge_index[1].reshape(nt, NCHUNK, C)
    lap2 = lap
    d_inv2 = d_inv[:, None]
    b0_2 = b0[None, :]
    b2_2 = b2[None, :]

    h = x
    for w, b in ((W0, b0_2), (W0, b0_2), (W0, b0_2), (W2, b2_2)):
        parts = _sc_aggregate(h, src2, dst2, lap2)
        h = _tc_update(h, parts, d_inv2, w, b)
    return h
